# Initial kernel scaffold; baseline (speedup 1.0000x reference)
import jax
import jax.numpy as jnp
from jax.experimental import pallas as pl


def kernel(edge_attr, emb0, emb1, emb2):
    raise NotImplementedError("write your pallas kernel here")


# speedup vs baseline: 1.0814x; 1.0814x over previous
import functools
import math

import jax
import jax.numpy as jnp
from jax import lax
from jax.experimental import pallas as pl
from jax.experimental.pallas import tpu as pltpu
from jax.experimental.pallas import tpu_sc as plsc

D1, D2, D3 = 5, 6, 2
EMB = 128
N_EDGES = 320000
NUM_COMBO = D1 * D2 * D3
TBL_ROWS = 64

NUM_CORES = 2
NUM_SUBCORES = 16
NUM_WORKERS = NUM_CORES * NUM_SUBCORES
CHUNK = N_EDGES // NUM_WORKERS
SUB = 80
NUM_SUB = CHUNK // SUB


def _prep_body(a_ref, b_ref, c_ref, e0_ref, e1_ref, e2_ref, combo_ref, tbl_ref):
    combo_ref[...] = a_ref[...] * (D2 * D3) + b_ref[...] * D3 + c_ref[...]
    r = lax.broadcasted_iota(jnp.int32, (TBL_ROWS, EMB), 0)
    a_i = r // (D2 * D3)
    b_i = (r // D3) % D2
    c_i = r % D3
    acc = jnp.zeros((TBL_ROWS, EMB), jnp.float32)
    for k in range(D1):
        acc = acc + jnp.where(a_i == k, e0_ref[pl.ds(k, 1), :], 0.0)
    for k in range(D2):
        acc = acc + jnp.where(b_i == k, e1_ref[pl.ds(k, 1), :], 0.0)
    for k in range(D3):
        acc = acc + jnp.where(c_i == k, e2_ref[pl.ds(k, 1), :], 0.0)
    tbl_ref[...] = acc * (1.0 / math.sqrt(3.0))


_prep = pl.pallas_call(
    _prep_body,
    out_shape=[
        jax.ShapeDtypeStruct((N_EDGES // EMB, EMB), jnp.int32),
        jax.ShapeDtypeStruct((TBL_ROWS, EMB), jnp.float32),
    ],
)


@functools.partial(
    pl.kernel,
    mesh=plsc.VectorSubcoreMesh(core_axis_name="c", subcore_axis_name="s"),
    out_type=jax.ShapeDtypeStruct((N_EDGES, EMB), jnp.float32),
    scratch_types=[
        pltpu.VMEM((SUB,), jnp.int32),
        pltpu.VMEM((SUB, EMB), jnp.float32),
        pltpu.SemaphoreType.DMA,
    ],
)
def _sc_gather(combo_hbm, tbl_hbm, out_hbm, idx_v, rows_v, sem):
    wid = lax.axis_index("s") * NUM_CORES + lax.axis_index("c")
    base = wid * CHUNK

    def body(i, carry):
        start = base + i * SUB
        pltpu.sync_copy(combo_hbm.at[pl.ds(start, SUB)], idx_v)
        pltpu.async_copy(tbl_hbm.at[idx_v], rows_v, sem).wait()
        pltpu.sync_copy(rows_v, out_hbm.at[pl.ds(start, SUB), :])
        return carry

    lax.fori_loop(0, NUM_SUB, body, 0)


def kernel(edge_attr, emb0, emb1, emb2):
    ea = edge_attr.astype(jnp.int32)
    a2 = ea[:, 0].reshape(N_EDGES // EMB, EMB)
    b2 = ea[:, 1].reshape(N_EDGES // EMB, EMB)
    c2 = ea[:, 2].reshape(N_EDGES // EMB, EMB)
    combo2, tbl = _prep(a2, b2, c2, emb0, emb1, emb2)
    combo = combo2.reshape(N_EDGES)
    return _sc_gather(combo, tbl)


# speedup vs baseline: 1.0984x; 1.0157x over previous
import functools
import math

import jax
import jax.numpy as jnp
from jax import lax
from jax.experimental import pallas as pl
from jax.experimental.pallas import tpu as pltpu
from jax.experimental.pallas import tpu_sc as plsc

D1, D2, D3 = 5, 6, 2
EMB = 128
N_EDGES = 320000
NUM_COMBO = D1 * D2 * D3
TBL_ROWS = 64

NUM_CORES = 2
NUM_SUBCORES = 16
NUM_WORKERS = NUM_CORES * NUM_SUBCORES
CHUNK = N_EDGES // NUM_WORKERS
SUB = 1000
NUM_SUB = CHUNK // SUB


def _prep_body(a_ref, b_ref, c_ref, e0_ref, e1_ref, e2_ref, combo_ref, tbl_ref):
    combo_ref[...] = a_ref[...] * (D2 * D3) + b_ref[...] * D3 + c_ref[...]
    r = lax.broadcasted_iota(jnp.int32, (TBL_ROWS, EMB), 0)
    a_i = r // (D2 * D3)
    b_i = (r // D3) % D2
    c_i = r % D3
    acc = jnp.zeros((TBL_ROWS, EMB), jnp.float32)
    for k in range(D1):
        acc = acc + jnp.where(a_i == k, e0_ref[pl.ds(k, 1), :], 0.0)
    for k in range(D2):
        acc = acc + jnp.where(b_i == k, e1_ref[pl.ds(k, 1), :], 0.0)
    for k in range(D3):
        acc = acc + jnp.where(c_i == k, e2_ref[pl.ds(k, 1), :], 0.0)
    tbl_ref[...] = acc * (1.0 / math.sqrt(3.0))


_prep = pl.pallas_call(
    _prep_body,
    out_shape=[
        jax.ShapeDtypeStruct((N_EDGES // EMB, EMB), jnp.int32),
        jax.ShapeDtypeStruct((TBL_ROWS, EMB), jnp.float32),
    ],
)


@functools.partial(
    pl.kernel,
    mesh=plsc.VectorSubcoreMesh(core_axis_name="c", subcore_axis_name="s"),
    out_type=jax.ShapeDtypeStruct((N_EDGES, EMB), jnp.float32),
    scratch_types=[
        pltpu.VMEM((SUB,), jnp.int32),
        pltpu.VMEM((SUB, EMB), jnp.float32),
        pltpu.SemaphoreType.DMA,
    ],
)
def _sc_gather(combo_hbm, tbl_hbm, out_hbm, idx_v, rows_v, sem):
    wid = lax.axis_index("s") * NUM_CORES + lax.axis_index("c")
    base = wid * CHUNK

    def body(i, carry):
        start = base + i * SUB
        pltpu.sync_copy(combo_hbm.at[pl.ds(start, SUB)], idx_v)
        pltpu.async_copy(tbl_hbm.at[idx_v], rows_v, sem).wait()
        pltpu.sync_copy(rows_v, out_hbm.at[pl.ds(start, SUB), :])
        return carry

    lax.fori_loop(0, NUM_SUB, body, 0)


def kernel(edge_attr, emb0, emb1, emb2):
    ea = edge_attr.astype(jnp.int32)
    a2 = ea[:, 0].reshape(N_EDGES // EMB, EMB)
    b2 = ea[:, 1].reshape(N_EDGES // EMB, EMB)
    c2 = ea[:, 2].reshape(N_EDGES // EMB, EMB)
    combo2, tbl = _prep(a2, b2, c2, emb0, emb1, emb2)
    combo = combo2.reshape(N_EDGES)
    return _sc_gather(combo, tbl)


# speedup vs baseline: 16.1737x; 14.7245x over previous
import functools
import math

import jax
import jax.numpy as jnp
from jax import lax
from jax.experimental import pallas as pl
from jax.experimental.pallas import tpu as pltpu
from jax.experimental.pallas import tpu_sc as plsc

D1, D2, D3 = 5, 6, 2
EMB = 128
N_EDGES = 320000
NUM_COMBO = D1 * D2 * D3
TBL_ROWS = 64

NUM_CORES = 2
NUM_SUBCORES = 16
NUM_WORKERS = NUM_CORES * NUM_SUBCORES
CHUNK = N_EDGES // NUM_WORKERS
SUB = 400
NUM_SUB = CHUNK // SUB


def _prep_body(a_ref, b_ref, c_ref, e0_ref, e1_ref, e2_ref, combo_ref, tbl_ref):
    combo_ref[...] = a_ref[...] * (D2 * D3) + b_ref[...] * D3 + c_ref[...]
    r = lax.broadcasted_iota(jnp.int32, (TBL_ROWS, EMB), 0)
    a_i = r // (D2 * D3)
    b_i = (r // D3) % D2
    c_i = r % D3
    acc = jnp.zeros((TBL_ROWS, EMB), jnp.float32)
    for k in range(D1):
        acc = acc + jnp.where(a_i == k, e0_ref[pl.ds(k, 1), :], 0.0)
    for k in range(D2):
        acc = acc + jnp.where(b_i == k, e1_ref[pl.ds(k, 1), :], 0.0)
    for k in range(D3):
        acc = acc + jnp.where(c_i == k, e2_ref[pl.ds(k, 1), :], 0.0)
    tbl_ref[...] = acc * (1.0 / math.sqrt(3.0))


_prep = pl.pallas_call(
    _prep_body,
    out_shape=[
        jax.ShapeDtypeStruct((N_EDGES // EMB, EMB), jnp.int32),
        jax.ShapeDtypeStruct((TBL_ROWS, EMB), jnp.float32),
    ],
)


@functools.partial(
    pl.kernel,
    mesh=plsc.VectorSubcoreMesh(core_axis_name="c", subcore_axis_name="s"),
    compiler_params=pltpu.CompilerParams(needs_layout_passes=False),
    out_type=jax.ShapeDtypeStruct((N_EDGES * EMB,), jnp.float32),
    scratch_types=[
        pltpu.VMEM((TBL_ROWS * EMB,), jnp.float32),
        pltpu.VMEM((SUB,), jnp.int32),
        pltpu.VMEM((SUB,), jnp.int32),
        pltpu.VMEM((SUB * EMB,), jnp.float32),
        pltpu.VMEM((SUB * EMB,), jnp.float32),
        pltpu.SemaphoreType.DMA,
        pltpu.SemaphoreType.DMA,
        pltpu.SemaphoreType.DMA,
        pltpu.SemaphoreType.DMA,
    ],
)
def _sc_expand(combo_hbm, tbl_hbm, out_hbm, tbl_v, idx0, idx1,
               out0, out1, isem0, isem1, wsem0, wsem1):
    wid = lax.axis_index("s") * NUM_CORES + lax.axis_index("c")
    base = wid * CHUNK
    idx_v = (idx0, idx1)
    out_v = (out0, out1)
    isem = (isem0, isem1)
    wsem = (wsem0, wsem1)

    pltpu.sync_copy(tbl_hbm, tbl_v)

    def expand(k):
        src = idx_v[k]
        dst = out_v[k]

        @plsc.parallel_loop(0, SUB // 16, unroll=1)
        def body(g):
            iv = src[pl.ds(g * 16, 16)] * EMB
            offs = [iv[l] for l in range(16)]
            gb = g * (16 * EMB)
            for l in range(16):
                eb = gb + l * EMB
                off = offs[l]
                for j in range(EMB // 16):
                    dst[pl.ds(eb + j * 16, 16)] = tbl_v[pl.ds(off + j * 16, 16)]

    def idx_dma_at(i, k):
        return pltpu.make_async_copy(
            combo_hbm.at[pl.ds(base + i * SUB, SUB)], idx_v[k], isem[k])

    def out_dma_at(i, k):
        return pltpu.make_async_copy(
            out_v[k], out_hbm.at[pl.ds((base + i * SUB) * EMB, SUB * EMB)],
            wsem[k])

    def process(i, k):
        idx_dma_at(i, k).wait()

        @pl.when(i >= 2)
        def _():
            out_dma_at(i - 2, k).wait()

        expand(k)
        out_dma_at(i, k).start()

        @pl.when(i + 2 < NUM_SUB)
        def _():
            idx_dma_at(i + 2, k).start()

    idx_dma_at(0, 0).start()
    idx_dma_at(1, 1).start()

    def pair_body(p, carry):
        process(2 * p, 0)
        process(2 * p + 1, 1)
        return carry

    lax.fori_loop(0, NUM_SUB // 2, pair_body, 0)
    process(NUM_SUB - 1, (NUM_SUB - 1) % 2)
    out_dma_at(NUM_SUB - 2, (NUM_SUB - 2) % 2).wait()
    out_dma_at(NUM_SUB - 1, (NUM_SUB - 1) % 2).wait()


def kernel(edge_attr, emb0, emb1, emb2):
    ea = edge_attr.astype(jnp.int32)
    a2 = ea[:, 0].reshape(N_EDGES // EMB, EMB)
    b2 = ea[:, 1].reshape(N_EDGES // EMB, EMB)
    c2 = ea[:, 2].reshape(N_EDGES // EMB, EMB)
    combo2, tbl = _prep(a2, b2, c2, emb0, emb1, emb2)
    combo = combo2.reshape(N_EDGES)
    tbl_flat = tbl.reshape(TBL_ROWS * EMB)
    out_flat = _sc_expand(combo, tbl_flat)
    return out_flat.reshape(N_EDGES, EMB)


# speedup vs baseline: 17.8744x; 1.1052x over previous
import functools
import math

import jax
import jax.numpy as jnp
from jax import lax
from jax.experimental import pallas as pl
from jax.experimental.pallas import tpu as pltpu
from jax.experimental.pallas import tpu_sc as plsc

D1, D2, D3 = 5, 6, 2
EMB = 128
N_EDGES = 320000
NUM_COMBO = D1 * D2 * D3
TBL_ROWS = 64

NUM_CORES = 2
NUM_SUBCORES = 16
NUM_WORKERS = NUM_CORES * NUM_SUBCORES
CHUNK = N_EDGES // NUM_WORKERS
SUB = 400
NUM_SUB = CHUNK // SUB


def _tbl_body(e0_ref, e1_ref, e2_ref, tbl_ref):
    r = lax.broadcasted_iota(jnp.int32, (TBL_ROWS, EMB), 0)
    a_i = r // (D2 * D3)
    b_i = (r // D3) % D2
    c_i = r % D3
    acc = jnp.zeros((TBL_ROWS, EMB), jnp.float32)
    for k in range(D1):
        acc = acc + jnp.where(a_i == k, e0_ref[pl.ds(k, 1), :], 0.0)
    for k in range(D2):
        acc = acc + jnp.where(b_i == k, e1_ref[pl.ds(k, 1), :], 0.0)
    for k in range(D3):
        acc = acc + jnp.where(c_i == k, e2_ref[pl.ds(k, 1), :], 0.0)
    tbl_ref[...] = acc * (1.0 / math.sqrt(3.0))


_tbl = pl.pallas_call(
    _tbl_body,
    out_shape=jax.ShapeDtypeStruct((TBL_ROWS, EMB), jnp.float32),
)


@functools.partial(
    pl.kernel,
    mesh=plsc.VectorSubcoreMesh(core_axis_name="c", subcore_axis_name="s"),
    compiler_params=pltpu.CompilerParams(needs_layout_passes=False),
    out_type=jax.ShapeDtypeStruct((N_EDGES * EMB,), jnp.float32),
    scratch_types=[
        pltpu.VMEM((TBL_ROWS * EMB,), jnp.float32),
        pltpu.VMEM((3 * SUB,), jnp.int32),
        pltpu.VMEM((3 * SUB,), jnp.int32),
        pltpu.VMEM((SUB * EMB,), jnp.float32),
        pltpu.VMEM((SUB * EMB,), jnp.float32),
        pltpu.SemaphoreType.DMA,
        pltpu.SemaphoreType.DMA,
        pltpu.SemaphoreType.DMA,
        pltpu.SemaphoreType.DMA,
    ],
)
def _sc_expand(eat_hbm, tbl_hbm, out_hbm, tbl_v, idx0, idx1,
               out0, out1, isem0, isem1, wsem0, wsem1):
    wid = lax.axis_index("s") * NUM_CORES + lax.axis_index("c")
    base = wid * CHUNK
    idx_v = (idx0, idx1)
    out_v = (out0, out1)
    isem = (isem0, isem1)
    wsem = (wsem0, wsem1)

    pltpu.sync_copy(tbl_hbm, tbl_v)

    def expand(k):
        src = idx_v[k]
        dst = out_v[k]

        @plsc.parallel_loop(0, SUB // 16, unroll=1)
        def body(g):
            av = src[pl.ds(g * 16, 16)]
            bv = src[pl.ds(SUB + g * 16, 16)]
            cv = src[pl.ds(2 * SUB + g * 16, 16)]
            iv = av * (D2 * D3 * EMB) + bv * (D3 * EMB) + cv * EMB
            offs = [iv[l] for l in range(16)]
            gb = g * (16 * EMB)
            for l in range(16):
                eb = gb + l * EMB
                off = offs[l]
                for j in range(EMB // 16):
                    dst[pl.ds(eb + j * 16, 16)] = tbl_v[pl.ds(off + j * 16, 16)]

    def idx_dmas_at(i, k):
        return [
            pltpu.make_async_copy(
                eat_hbm.at[pl.ds(f * N_EDGES + base + i * SUB, SUB)],
                idx_v[k].at[pl.ds(f * SUB, SUB)],
                isem[k])
            for f in range(3)
        ]

    def out_dma_at(i, k):
        return pltpu.make_async_copy(
            out_v[k], out_hbm.at[pl.ds((base + i * SUB) * EMB, SUB * EMB)],
            wsem[k])

    def process(i, k):
        for d in idx_dmas_at(i, k):
            d.wait()

        @pl.when(i >= 2)
        def _():
            out_dma_at(i - 2, k).wait()

        expand(k)
        out_dma_at(i, k).start()

        @pl.when(i + 2 < NUM_SUB)
        def _():
            for d in idx_dmas_at(i + 2, k):
                d.start()

    for d in idx_dmas_at(0, 0):
        d.start()
    for d in idx_dmas_at(1, 1):
        d.start()

    def pair_body(p, carry):
        process(2 * p, 0)
        process(2 * p + 1, 1)
        return carry

    lax.fori_loop(0, NUM_SUB // 2, pair_body, 0)
    process(NUM_SUB - 1, (NUM_SUB - 1) % 2)
    out_dma_at(NUM_SUB - 2, (NUM_SUB - 2) % 2).wait()
    out_dma_at(NUM_SUB - 1, (NUM_SUB - 1) % 2).wait()


def kernel(edge_attr, emb0, emb1, emb2):
    ea = edge_attr.astype(jnp.int32)
    eat = ea.T.reshape(3 * N_EDGES)
    tbl = _tbl(emb0, emb1, emb2)
    tbl_flat = tbl.reshape(TBL_ROWS * EMB)
    out_flat = _sc_expand(eat, tbl_flat)
    return out_flat.reshape(N_EDGES, EMB)


# speedup vs baseline: 17.9322x; 1.0032x over previous
import functools
import math

import jax
import jax.numpy as jnp
from jax import lax
from jax.experimental import pallas as pl
from jax.experimental.pallas import tpu as pltpu
from jax.experimental.pallas import tpu_sc as plsc

D1, D2, D3 = 5, 6, 2
EMB = 128
N_EDGES = 320000
NUM_COMBO = D1 * D2 * D3

NUM_CORES = 2
NUM_SUBCORES = 16
NUM_WORKERS = NUM_CORES * NUM_SUBCORES
CHUNK = N_EDGES // NUM_WORKERS
SUB = 400
NUM_SUB = CHUNK // SUB
SCALE = 1.0 / math.sqrt(3.0)


@functools.partial(
    pl.kernel,
    mesh=plsc.VectorSubcoreMesh(core_axis_name="c", subcore_axis_name="s"),
    compiler_params=pltpu.CompilerParams(needs_layout_passes=False),
    out_type=jax.ShapeDtypeStruct((N_EDGES * EMB,), jnp.float32),
    scratch_types=[
        pltpu.VMEM(((D1 + D2 + D3) * EMB,), jnp.float32),
        pltpu.VMEM((NUM_COMBO * EMB,), jnp.float32),
        pltpu.VMEM((3 * SUB,), jnp.int32),
        pltpu.VMEM((3 * SUB,), jnp.int32),
        pltpu.VMEM((SUB * EMB,), jnp.float32),
        pltpu.VMEM((SUB * EMB,), jnp.float32),
        pltpu.SemaphoreType.DMA,
        pltpu.SemaphoreType.DMA,
        pltpu.SemaphoreType.DMA,
        pltpu.SemaphoreType.DMA,
    ],
)
def _sc_expand(ea_hbm, emb_hbm, out_hbm, emb_v, tbl_v, idx0, idx1,
               out0, out1, isem0, isem1, wsem0, wsem1):
    wid = lax.axis_index("s") * NUM_CORES + lax.axis_index("c")
    base = wid * CHUNK
    idx_v = (idx0, idx1)
    out_v = (out0, out1)
    isem = (isem0, isem1)
    wsem = (wsem0, wsem1)

    def idx_dmas_at(i, k):
        return [
            pltpu.make_async_copy(
                ea_hbm.at[pl.ds(f * N_EDGES + base + i * SUB, SUB)],
                idx_v[k].at[pl.ds(f * SUB, SUB)],
                isem[k])
            for f in range(3)
        ]

    def out_dma_at(i, k):
        return pltpu.make_async_copy(
            out_v[k], out_hbm.at[pl.ds((base + i * SUB) * EMB, SUB * EMB)],
            wsem[k])

    for d in idx_dmas_at(0, 0):
        d.start()
    for d in idx_dmas_at(1, 1):
        d.start()

    pltpu.sync_copy(emb_hbm, emb_v)
    NJ = EMB // 16
    for a in range(D1):
        sa = [emb_v[pl.ds(a * EMB + j * 16, 16)] * SCALE for j in range(NJ)]
        for b in range(D2):
            sb = [sa[j] + emb_v[pl.ds((D1 + b) * EMB + j * 16, 16)] * SCALE
                  for j in range(NJ)]
            for c in range(D3):
                r = (a * D2 + b) * D3 + c
                for j in range(NJ):
                    tbl_v[pl.ds(r * EMB + j * 16, 16)] = (
                        sb[j]
                        + emb_v[pl.ds((D1 + D2 + c) * EMB + j * 16, 16)] * SCALE
                    )

    def expand(k):
        src = idx_v[k]
        dst = out_v[k]

        @plsc.parallel_loop(0, SUB // 16, unroll=1)
        def body(g):
            av = src[pl.ds(g * 16, 16)]
            bv = src[pl.ds(SUB + g * 16, 16)]
            cv = src[pl.ds(2 * SUB + g * 16, 16)]
            iv = av * (D2 * D3 * EMB) + bv * (D3 * EMB) + cv * EMB
            offs = [iv[l] for l in range(16)]
            gb = g * (16 * EMB)
            for l in range(16):
                eb = gb + l * EMB
                off = offs[l]
                for j in range(EMB // 16):
                    dst[pl.ds(eb + j * 16, 16)] = tbl_v[pl.ds(off + j * 16, 16)]

    def process(i, k):
        for d in idx_dmas_at(i, k):
            d.wait()

        @pl.when(i >= 2)
        def _():
            out_dma_at(i - 2, k).wait()

        expand(k)
        out_dma_at(i, k).start()

        @pl.when(i + 2 < NUM_SUB)
        def _():
            for d in idx_dmas_at(i + 2, k):
                d.start()

    def pair_body(p, carry):
        process(2 * p, 0)
        process(2 * p + 1, 1)
        return carry

    lax.fori_loop(0, NUM_SUB // 2, pair_body, 0)
    process(NUM_SUB - 1, (NUM_SUB - 1) % 2)
    out_dma_at(NUM_SUB - 2, (NUM_SUB - 2) % 2).wait()
    out_dma_at(NUM_SUB - 1, (NUM_SUB - 1) % 2).wait()


def kernel(edge_attr, emb0, emb1, emb2):
    eat = edge_attr.astype(jnp.int32).T.reshape(3 * N_EDGES)
    emb = jnp.concatenate([emb0, emb1, emb2], axis=0).reshape(-1)
    out_flat = _sc_expand(eat, emb)
    return out_flat.reshape(N_EDGES, EMB)
